# baseline (speedup 1.0000x reference)
import jax
import jax.numpy as jnp
from jax.experimental import pallas as pl
from jax.experimental.pallas import tpu as pltpu

EPS_NORM = 1e-06
EPS_MEAN = 1e-06

_S = 2048
_C = 6
_BB = 128


def _sh1(v):
    return jnp.concatenate([v[:, 1:], v[:, :1]], axis=1)


def _sig_block(a_ref, out_ref):
    f32 = jnp.float32
    a6 = a_ref[...].reshape(_C * _BB, _S)
    x = a6[0:_BB]
    y = a6[_BB:2 * _BB]
    u = a6[2 * _BB:]

    ones = jnp.ones((_S, 1), f32)

    def rs(v):
        return jnp.dot(v, ones, preferred_element_type=f32)

    w1 = _sh1(a6[0:2 * _BB])
    x1 = w1[0:_BB]
    y1 = w1[_BB:]

    u16 = u.astype(jnp.bfloat16)
    du = jnp.abs(_sh1(u16) - u16)
    s_du = (jnp.dot(du, jnp.ones((_S, 1), jnp.bfloat16),
                    preferred_element_type=f32)
            - du[:, _S - 1:_S].astype(f32))

    bf16 = jnp.bfloat16
    q16 = u16 * u16
    q4 = q16.reshape(2, 2, _BB, _S)
    pf2 = (q4[:, 0] + q4[:, 1]).reshape(2 * _BB, _S)
    pf1 = pf2 * jax.lax.rsqrt(jnp.maximum(pf2, bf16(1e-30)))

    cr = x * y1 - y * x1
    p0 = x * x + y * y
    pn = p0 * (x1 * x1 + y1 * y1)
    okf = (pn > EPS_NORM * EPS_NORM).astype(f32)
    cv = jnp.abs(cr) * jax.lax.rsqrt(jnp.maximum(pn, EPS_NORM * EPS_NORM))
    cv = cv * okf

    def edge2(v):
        return v[:, 0:1] + v[:, _S - 1:_S]

    scurv = rs(cv) - edge2(cv)
    scnt = rs(okf) - edge2(okf)

    sd_vel = s_du[0:_BB] + s_du[_BB:2 * _BB]
    sd_acc = s_du[2 * _BB:3 * _BB] + s_du[3 * _BB:]
    ones16 = jnp.ones((_S, 1), bf16)
    s1 = jnp.dot(pf1, ones16, preferred_element_type=f32)
    s2 = jnp.dot(pf2, ones16, preferred_element_type=f32)
    ss1, sf1 = s1[0:_BB], s1[_BB:]
    ss2, sf2 = s2[0:_BB], s2[_BB:]

    pc = jnp.where(scnt > 0, scurv / jnp.maximum(scnt, 1.0), 0.0)
    vs = 1.0 / (1.0 + sd_vel * (1.0 / (2 * (_S - 1))))
    aj = sd_acc * (1.0 / (2 * (_S - 1)))

    mean_s = ss1 * (1.0 / _S)
    var_s = jnp.maximum(ss2 * (1.0 / _S) - mean_s * mean_s, 0.0)
    mr = jnp.sqrt(var_s) / (mean_s + EPS_MEAN)

    mean_f = sf1 * (1.0 / _S)
    var_f = jnp.maximum(sf2 * (1.0 / _S) - mean_f * mean_f, 0.0)
    fm = jnp.sqrt(var_f) / (mean_f + EPS_MEAN)

    out_ref[...] = jnp.concatenate(
        [v.reshape(1, _BB) for v in (pc, vs, aj, mr, fm)], axis=0)


@jax.jit
def kernel(trajectories):
    b = trajectories.shape[0]
    at = jnp.transpose(trajectories, (2, 0, 1))
    grid = (b // _BB,)
    return pl.pallas_call(
        _sig_block,
        grid=grid,
        in_specs=[pl.BlockSpec((_C, _BB, _S), lambda i: (0, i, 0))],
        out_specs=pl.BlockSpec((5, _BB), lambda i: (0, i)),
        out_shape=jax.ShapeDtypeStruct((5, b), jnp.float32),
        compiler_params=pltpu.CompilerParams(
            dimension_semantics=("arbitrary",),
        ),
    )(at).T


# speedup vs baseline: 1.0121x; 1.0121x over previous
import jax
import jax.numpy as jnp
from jax.experimental import pallas as pl
from jax.experimental.pallas import tpu as pltpu

EPS_NORM = 1e-06
EPS_MEAN = 1e-06

_S = 2048
_C = 6
_BB = 128


def _sh1(v):
    return jnp.concatenate([v[:, 1:], v[:, :1]], axis=1)


def _sig_block(a_ref, out_ref):
    f32 = jnp.float32
    a6 = a_ref[...].reshape(_C * _BB, _S)
    x = a6[0:_BB]
    y = a6[_BB:2 * _BB]
    u = a6[2 * _BB:]

    w1 = _sh1(a6[0:2 * _BB])
    x1 = w1[0:_BB]
    y1 = w1[_BB:]

    u16 = u.astype(jnp.bfloat16)
    du = jnp.abs(_sh1(u16) - u16)
    s_du = (jnp.dot(du, jnp.ones((_S, 1), jnp.bfloat16),
                    preferred_element_type=f32)
            - du[:, _S - 1:_S].astype(f32))

    bf16 = jnp.bfloat16
    q16 = u16 * u16
    q4 = q16.reshape(2, 2, _BB, _S)
    pf2 = (q4[:, 0] + q4[:, 1]).reshape(2 * _BB, _S)
    pf1 = pf2 * jax.lax.rsqrt(jnp.maximum(pf2, bf16(1e-30)))

    cr = x * y1 - y * x1
    p0 = x * x + y * y
    pn = p0 * (x1 * x1 + y1 * y1)
    okf = (pn > EPS_NORM * EPS_NORM).astype(f32)
    cv = jnp.abs(cr) * jax.lax.rsqrt(jnp.maximum(pn, EPS_NORM * EPS_NORM))
    cv = cv * okf

    ones = jnp.ones((_S, 1), f32)

    def rs(v):
        return jnp.dot(v, ones, preferred_element_type=f32)

    def edge2(v):
        return v[:, 0:1] + v[:, _S - 1:_S]

    scurv = rs(cv) - edge2(cv)
    scnt = rs(okf) - edge2(okf)

    sd_vel = s_du[0:_BB] + s_du[_BB:2 * _BB]
    sd_acc = s_du[2 * _BB:3 * _BB] + s_du[3 * _BB:]
    ones16 = jnp.ones((_S, 1), bf16)
    s1 = jnp.dot(pf1, ones16, preferred_element_type=f32)
    s2 = jnp.dot(pf2, ones16, preferred_element_type=f32)
    ss1, sf1 = s1[0:_BB], s1[_BB:]
    ss2, sf2 = s2[0:_BB], s2[_BB:]

    pc = jnp.where(scnt > 0, scurv / jnp.maximum(scnt, 1.0), 0.0)
    vs = 1.0 / (1.0 + sd_vel * (1.0 / (2 * (_S - 1))))
    aj = sd_acc * (1.0 / (2 * (_S - 1)))

    mean_s = ss1 * (1.0 / _S)
    var_s = jnp.maximum(ss2 * (1.0 / _S) - mean_s * mean_s, 0.0)
    mr = jnp.sqrt(var_s) / (mean_s + EPS_MEAN)

    mean_f = sf1 * (1.0 / _S)
    var_f = jnp.maximum(sf2 * (1.0 / _S) - mean_f * mean_f, 0.0)
    fm = jnp.sqrt(var_f) / (mean_f + EPS_MEAN)

    out_ref[...] = jnp.concatenate(
        [v.reshape(1, _BB) for v in (pc, vs, aj, mr, fm)], axis=0)


@jax.jit
def kernel(trajectories):
    b = trajectories.shape[0]
    at = jnp.transpose(trajectories, (2, 0, 1))
    grid = (b // _BB,)
    return pl.pallas_call(
        _sig_block,
        grid=grid,
        in_specs=[pl.BlockSpec((_C, _BB, _S), lambda i: (0, i, 0))],
        out_specs=pl.BlockSpec((5, _BB), lambda i: (0, i)),
        out_shape=jax.ShapeDtypeStruct((5, b), jnp.float32),
        compiler_params=pltpu.CompilerParams(
            dimension_semantics=("arbitrary",),
        ),
    )(at).T
